# baseline (speedup 1.0000x reference)
import jax
import jax.numpy as jnp
from jax.experimental import pallas as pl


def _fused_gcn(ei_ref, fea_ref, w1_ref, b1_ref, w2_ref, b2_ref, out_ref):
    ei = ei_ref[...]
    n = fea_ref.shape[0]
    e = ei.shape[1]
    f32 = jnp.float32

    node_iota = jax.lax.broadcasted_iota(ei.dtype, (n, e), 0)
    sm = (node_iota == ei[0:1, :]).astype(f32)
    dm = (node_iota == ei[1:2, :]).astype(f32)

    deg = 1.0 + jnp.sum(dm, axis=1, keepdims=True)
    dinv = jax.lax.rsqrt(deg)

    dsrc = jnp.sum(sm * dinv, axis=0, keepdims=True)
    ddst = jnp.sum(dm * dinv, axis=0, keepdims=True)
    norm = dsrc * ddst

    a = jax.lax.dot_general(dm * norm, sm, (((1,), (1,)), ((), ())),
                            preferred_element_type=f32)
    ii = jax.lax.broadcasted_iota(jnp.int32, (n, n), 0)
    jj = jax.lax.broadcasted_iota(jnp.int32, (n, n), 1)
    a = a + (ii == jj).astype(f32) * (dinv * dinv)

    h1 = jnp.dot(fea_ref[...], w1_ref[...], preferred_element_type=f32)
    x1 = jnp.maximum(jnp.dot(a, h1, preferred_element_type=f32) + b1_ref[...], 0.0)
    h2 = jnp.dot(x1, w2_ref[...], preferred_element_type=f32)
    out_ref[...] = jnp.dot(a, h2, preferred_element_type=f32) + b2_ref[...]


def kernel(fea, edge_index, W1, b1, W2, b2):
    ei = edge_index.astype(jnp.int32)
    out = pl.pallas_call(
        _fused_gcn,
        out_shape=jax.ShapeDtypeStruct((fea.shape[0], W2.shape[1]), jnp.float32),
    )(ei, fea, W1, b1.reshape(1, -1), W2, b2.reshape(1, -1))
    return out
